# Initial kernel scaffold; baseline (speedup 1.0000x reference)
import jax
import jax.numpy as jnp
from jax.experimental import pallas as pl


def kernel(pred_logits, pred_boxes, target_sizes):
    raise NotImplementedError("write your pallas kernel here")


# speedup vs baseline: 3.0575x; 3.0575x over previous
import functools

import jax
import jax.numpy as jnp
import numpy as np
from jax import lax
from jax.experimental import pallas as pl
from jax.experimental.pallas import tpu as pltpu
from jax.experimental.pallas import tpu_sc as plsc

NQ = 300
NPATCH = 256
NPAD = 304
NCHUNK = NPAD // 16
KPAD = 112
KCHUNK = KPAD // 16
PRE = 100
KEEP = 20
OUTW = 160
PER_W = 8

_LANE = np.arange(16, dtype=np.int32)


def _nms_body(s_hbm, cx_hbm, cy_hbm, w_hbm, h_hbm, swsh_hbm, out_hbm,
              sbuf, cxbuf, cybuf, wbuf, hbuf, swshv,
              cs_r, ci_r, ss_r, sx1_r, sy1_r, sx2_r, sy2_r, ar_r,
              sup_r, pu_r, pv_r, stage_r):
    ncores = 2
    wid = lax.axis_index("s") * ncores + lax.axis_index("c")
    base = wid * PER_W

    pltpu.sync_copy(s_hbm.at[pl.ds(base * NPAD, PER_W * NPAD)], sbuf)
    pltpu.sync_copy(cx_hbm.at[pl.ds(base * NPAD, PER_W * NPAD)], cxbuf)
    pltpu.sync_copy(cy_hbm.at[pl.ds(base * NPAD, PER_W * NPAD)], cybuf)
    pltpu.sync_copy(w_hbm.at[pl.ds(base * NPAD, PER_W * NPAD)], wbuf)
    pltpu.sync_copy(h_hbm.at[pl.ds(base * NPAD, PER_W * NPAD)], hbuf)
    pltpu.sync_copy(swsh_hbm.at[pl.ds(wid * 16, 16)], swshv)

    lane = lax.iota(jnp.int32, 16)
    zeros_i = jnp.full((16,), 0, jnp.int32)

    def splat(ref, i):
        return plsc.load_gather(ref, [jnp.full((16,), i, jnp.int32)])

    def patch_body(k, _):
        off = k * NPAD
        swv = splat(swshv, k)
        shv = splat(swshv, k + 8)

        svs = [sbuf[pl.ds(off + 16 * c, 16)] for c in range(NCHUNK)]

        def bit_body(t, kbits):
            b = 30 - t
            trial = jnp.bitwise_or(kbits, lax.shift_left(jnp.int32(1), b))
            tv = plsc.bitcast(jnp.full((16,), trial, jnp.int32), jnp.float32)
            cnt = zeros_i
            for c in range(NCHUNK):
                cnt = cnt + jnp.where(svs[c] >= tv, 1, 0)
            return jnp.where(jnp.sum(cnt) >= PRE, trial, kbits)

        kbits = lax.fori_loop(0, 31, bit_body, jnp.int32(0))
        thv = plsc.bitcast(jnp.full((16,), kbits, jnp.int32), jnp.float32)

        gcnt = zeros_i
        for c in range(NCHUNK):
            gcnt = gcnt + jnp.where(svs[c] > thv, 1, 0)
        tie_budget = PRE - jnp.sum(gcnt)

        ci_r[pl.ds(96, 16)] = lane + NPAD
        cs_r[pl.ds(96, 16)] = jnp.full((16,), -1.0, jnp.float32)
        nsel = jnp.int32(0)
        eqrun = jnp.int32(0)
        for c in range(NCHUNK):
            sv = svs[c]
            gt = sv > thv
            eq = sv == thv
            eqi = jnp.where(eq, 1, 0)
            eqexc = plsc.cumsum(eqi) - eqi
            sel = jnp.logical_or(gt, jnp.logical_and(eq, (eqrun + eqexc) < tie_budget))
            seli = jnp.where(sel, 1, 0)
            dest = jnp.minimum(nsel + plsc.cumsum(seli) - seli, KPAD - 1)
            idxv = lane + 16 * c
            plsc.store_scatter(cs_r, [dest], sv, mask=sel)
            plsc.store_scatter(ci_r, [dest], idxv, mask=sel)
            nsel = nsel + jnp.sum(seli)
            eqrun = eqrun + jnp.sum(eqi)

        csv = [cs_r[pl.ds(16 * c, 16)] for c in range(KCHUNK)]
        civ = [ci_r[pl.ds(16 * c, 16)] for c in range(KCHUNK)]

        def rank_body(j, rk):
            sj = splat(cs_r, j)
            ij = splat(ci_r, j)
            out = []
            for c in range(KCHUNK):
                win = jnp.logical_or(
                    sj > csv[c],
                    jnp.logical_and(sj == csv[c], ij < civ[c]))
                out.append(rk[c] + jnp.where(win, 1, 0))
            return tuple(out)

        rank = lax.fori_loop(0, KPAD, rank_body, tuple([zeros_i] * KCHUNK))
        for c in range(KCHUNK):
            plsc.store_scatter(ss_r, [rank[c]], csv[c])
            plsc.store_scatter(pu_r, [rank[c]], civ[c])

        half = jnp.float32(0.5)
        for c in range(KCHUNK):
            gi = jnp.minimum(pu_r[pl.ds(16 * c, 16)], NPAD - 1) + off
            bcx = plsc.load_gather(cxbuf, [gi])
            bcy = plsc.load_gather(cybuf, [gi])
            bw = plsc.load_gather(wbuf, [gi])
            bh = plsc.load_gather(hbuf, [gi])
            x1 = (bcx - half * bw) * swv
            y1 = (bcy - half * bh) * shv
            x2 = (bcx + half * bw) * swv
            y2 = (bcy + half * bh) * shv
            sx1_r[pl.ds(16 * c, 16)] = x1
            sy1_r[pl.ds(16 * c, 16)] = y1
            sx2_r[pl.ds(16 * c, 16)] = x2
            sy2_r[pl.ds(16 * c, 16)] = y2
            ar_r[pl.ds(16 * c, 16)] = (x2 - x1) * (y2 - y1)
            sup_r[pl.ds(16 * c, 16)] = zeros_i

        x1v = [sx1_r[pl.ds(16 * c, 16)] for c in range(KCHUNK)]
        y1v = [sy1_r[pl.ds(16 * c, 16)] for c in range(KCHUNK)]
        x2v = [sx2_r[pl.ds(16 * c, 16)] for c in range(KCHUNK)]
        y2v = [sy2_r[pl.ds(16 * c, 16)] for c in range(KCHUNK)]
        arv = [ar_r[pl.ds(16 * c, 16)] for c in range(KCHUNK)]
        posv = [lane + 16 * c for c in range(KCHUNK)]
        thr = jnp.float32(0.7)
        eps = jnp.float32(1e-9)
        fzero = jnp.float32(0.0)

        def nms_body(i, sup):
            x1i = splat(sx1_r, i)
            y1i = splat(sy1_r, i)
            x2i = splat(sx2_r, i)
            y2i = splat(sy2_r, i)
            ari = splat(ar_r, i)
            actv = splat(sup_r, i) == 0
            iv = jnp.full((16,), i, jnp.int32)
            for c in range(KCHUNK):
                xx1 = jnp.maximum(x1i, x1v[c])
                yy1 = jnp.maximum(y1i, y1v[c])
                xx2 = jnp.minimum(x2i, x2v[c])
                yy2 = jnp.minimum(y2i, y2v[c])
                ww = jnp.maximum(xx2 - xx1, fzero)
                hh = jnp.maximum(yy2 - yy1, fzero)
                inter = ww * hh
                union = ari + arv[c] - inter
                iou = inter / (union + eps)
                cond = jnp.logical_and(
                    jnp.logical_and(iou > thr, posv[c] > iv), actv)
                supc = sup_r[pl.ds(16 * c, 16)]
                sup_r[pl.ds(16 * c, 16)] = jnp.bitwise_or(
                    supc, jnp.where(cond, 1, 0))
            return sup

        lax.fori_loop(0, PRE, nms_body, jnp.int32(0))

        ru = jnp.int32(0)
        rv = jnp.int32(0)
        for c in range(KCHUNK):
            supc = sup_r[pl.ds(16 * c, 16)]
            real = posv[c] < PRE
            u = jnp.where(jnp.logical_and(supc == 0, real), 1, 0)
            v = jnp.where(jnp.logical_and(supc != 0, real), 1, 0)
            pu_r[pl.ds(16 * c, 16)] = ru + plsc.cumsum(u) - u
            pv_r[pl.ds(16 * c, 16)] = rv + plsc.cumsum(v) - v
            ru = ru + jnp.sum(u)
            rv = rv + jnp.sum(v)
        uv = jnp.full((16,), ru, jnp.int32)
        for c in range(KCHUNK):
            supc = sup_r[pl.ds(16 * c, 16)]
            real = posv[c] < PRE
            unsup = jnp.logical_and(supc == 0, real)
            kpos = jnp.where(unsup, pu_r[pl.ds(16 * c, 16)],
                             uv + pv_r[pl.ds(16 * c, 16)])
            m20 = jnp.logical_and(kpos < KEEP, real)
            kcl = jnp.minimum(kpos, 31)
            plsc.store_scatter(stage_r, [kcl], ss_r[pl.ds(16 * c, 16)], mask=m20)
            plsc.store_scatter(stage_r, [kcl + 32], x1v[c], mask=m20)
            plsc.store_scatter(stage_r, [kcl + 64], y1v[c], mask=m20)
            plsc.store_scatter(stage_r, [kcl + 96], x2v[c], mask=m20)
            plsc.store_scatter(stage_r, [kcl + 128], y2v[c], mask=m20)

        pltpu.sync_copy(stage_r, out_hbm.at[pl.ds((base + k) * OUTW, OUTW)])
        return 0

    lax.fori_loop(0, PER_W, patch_body, 0)


@jax.jit
def kernel(pred_logits, pred_boxes, target_sizes):
    bs, n, _ = pred_logits.shape
    scores = jax.nn.sigmoid(pred_logits[..., -1]).reshape(NPATCH, NQ)
    s_pad = jnp.pad(scores, ((0, 0), (0, NPAD - NQ)), constant_values=-1.0)
    boxes = pred_boxes.reshape(NPATCH, NQ, 4)
    comps = [jnp.pad(boxes[..., i], ((0, 0), (0, NPAD - NQ))) for i in range(4)]

    img_h = target_sizes[:, 0]
    img_w = target_sizes[:, 1]
    sw = jnp.repeat(img_w, NPATCH // bs).reshape(32, 8)
    sh = jnp.repeat(img_h, NPATCH // bs).reshape(32, 8)
    swsh = jnp.concatenate([sw, sh], axis=1).reshape(-1)

    mesh = plsc.VectorSubcoreMesh(core_axis_name="c", subcore_axis_name="s",
                                  num_cores=2, num_subcores=16)
    run = pl.kernel(
        _nms_body,
        out_type=jax.ShapeDtypeStruct((NPATCH * OUTW,), jnp.float32),
        mesh=mesh,
        compiler_params=pltpu.CompilerParams(needs_layout_passes=False),
        scratch_types=[
            pltpu.VMEM((PER_W * NPAD,), jnp.float32),
            pltpu.VMEM((PER_W * NPAD,), jnp.float32),
            pltpu.VMEM((PER_W * NPAD,), jnp.float32),
            pltpu.VMEM((PER_W * NPAD,), jnp.float32),
            pltpu.VMEM((PER_W * NPAD,), jnp.float32),
            pltpu.VMEM((16,), jnp.float32),
            pltpu.VMEM((KPAD,), jnp.float32),
            pltpu.VMEM((KPAD,), jnp.int32),
            pltpu.VMEM((KPAD,), jnp.float32),
            pltpu.VMEM((KPAD,), jnp.float32),
            pltpu.VMEM((KPAD,), jnp.float32),
            pltpu.VMEM((KPAD,), jnp.float32),
            pltpu.VMEM((KPAD,), jnp.float32),
            pltpu.VMEM((KPAD,), jnp.float32),
            pltpu.VMEM((KPAD,), jnp.int32),
            pltpu.VMEM((KPAD,), jnp.int32),
            pltpu.VMEM((KPAD,), jnp.int32),
            pltpu.VMEM((OUTW,), jnp.float32),
        ],
    )
    flat = run(s_pad.reshape(-1), comps[0].reshape(-1), comps[1].reshape(-1),
               comps[2].reshape(-1), comps[3].reshape(-1), swsh)
    out = flat.reshape(NPATCH, 5, 32)[:, :, :KEEP]
    return out.transpose(0, 2, 1).reshape(bs, NPATCH // bs, KEEP, 5)
